# Initial kernel scaffold; baseline (speedup 1.0000x reference)
import jax
import jax.numpy as jnp
from jax.experimental import pallas as pl


def kernel(x, edge_index, W1, att_src1, att_dst1, b1, W2, att_src2, att_dst2, b2):
    raise NotImplementedError("write your pallas kernel here")


# speedup vs baseline: 4.1369x; 4.1369x over previous
import functools

import jax
import jax.numpy as jnp
from jax import lax
from jax.experimental import pallas as pl
from jax.experimental.pallas import tpu as pltpu
from jax.experimental.pallas import tpu_sc as plsc

N = 10000
E_RAW = 160000
E_TOT = E_RAW + N
NW = 32
CH = 64
CHUNKS = -(-E_TOT // (NW * CH))
E_PAD = NW * CH * CHUNKS
ACC_ROWS = 10240
ZB = ACC_ROWS // 16
HEADS = 8
HID = 128
NB = 400
GRID = N // NB


def _mesh():
    return plsc.VectorSubcoreMesh(core_axis_name="c", subcore_axis_name="s",
                                  num_cores=2, num_subcores=16)


def _make_edge_kernel(D, with_msg):
    def body(src_h, dstg_h, dst_h, ta_h, tb_h, tm_h, out_h,
             isrc, idg, idst, av, bv, mv, val, acc, sem):
        c = lax.axis_index("c")
        s = lax.axis_index("s")
        wid = s * 2 + c
        slab = max(1, 256 // D)

        def zfill(r, carry):
            val[pl.ds(r * slab, slab), :] = jnp.zeros((slab, D), jnp.float32)
            return carry

        lax.fori_loop(0, CH // slab, zfill, 0)

        def zero_strip(j, carry):
            pltpu.sync_copy(val, acc.at[pl.ds(s * ZB + j * CH, CH)])
            return carry

        lax.fori_loop(0, ZB // CH, zero_strip, 0)
        plsc.subcore_barrier()

        def chunk(k, carry):
            off = (wid * CHUNKS + k) * CH
            pltpu.sync_copy(src_h.at[pl.ds(off, CH)], isrc)
            pltpu.sync_copy(dstg_h.at[pl.ds(off, CH)], idg)
            pltpu.sync_copy(dst_h.at[pl.ds(off, CH)], idst)
            pltpu.async_copy(ta_h.at[isrc], av, sem).wait()
            pltpu.async_copy(tb_h.at[idg], bv, sem).wait()
            if with_msg:
                pltpu.async_copy(tm_h.at[isrc], mv, sem).wait()

            def compute(r, carry2):
                rs = pl.ds(r * slab, slab)
                a = av[rs, :] + bv[rs, :]
                w = jnp.exp(jnp.where(a >= 0, a, 0.2 * a))
                if with_msg:
                    val[rs, :] = w * mv[rs, :]
                else:
                    val[rs, :] = w
                return carry2

            lax.fori_loop(0, CH // slab, compute, 0)
            pltpu.sync_copy(val, acc.at[idst], add=True)
            return carry

        lax.fori_loop(0, CHUNKS, chunk, 0)
        plsc.subcore_barrier()

        @pl.when(s == 0)
        def _():
            pltpu.sync_copy(acc, out_h.at[c])

    def no_msg_body(src_h, dstg_h, dst_h, ta_h, tb_h, out_h,
                    isrc, idg, idst, av, bv, val, acc, sem):
        body(src_h, dstg_h, dst_h, ta_h, tb_h, None, out_h,
             isrc, idg, idst, av, bv, None, val, acc, sem)

    scratch = [
        pltpu.VMEM((CH,), jnp.int32),
        pltpu.VMEM((CH,), jnp.int32),
        pltpu.VMEM((CH,), jnp.int32),
        pltpu.VMEM((CH, D), jnp.float32),
        pltpu.VMEM((CH, D), jnp.float32),
    ]
    if with_msg:
        scratch.append(pltpu.VMEM((CH, D), jnp.float32))
    scratch += [
        pltpu.VMEM((CH, D), jnp.float32),
        pltpu.VMEM_SHARED((ACC_ROWS, D), jnp.float32),
        pltpu.SemaphoreType.DMA,
    ]
    return pl.kernel(
        body if with_msg else no_msg_body,
        out_type=jax.ShapeDtypeStruct((2, ACC_ROWS, D), jnp.float32),
        mesh=_mesh(),
        scratch_types=scratch,
        compiler_params=pltpu.CompilerParams(use_tc_tiling_on_sc=False),
    )


def _k1_body(x_ref, w_ref, am_ref, bm_ref, h_ref, as_ref, ad_ref):
    h = jnp.dot(x_ref[...], w_ref[...], preferred_element_type=jnp.float32)
    h_ref[...] = h
    as_ref[...] = jnp.dot(h, am_ref[...], preferred_element_type=jnp.float32)
    ad_ref[...] = jnp.dot(h, bm_ref[...], preferred_element_type=jnp.float32)


def _k4_body(m_ref, d_ref, b1_ref, w2_ref, as2_ref, ad2_ref,
             h2_ref, a2s_ref, a2d_ref):
    den = d_ref[0] + d_ref[1]
    acc = jnp.zeros((NB, 3), jnp.float32)
    for h in range(HEADS):
        msum = m_ref[h, 0] + m_ref[h, 1]
        g = msum / (den[:, h:h + 1] + 1e-16) + b1_ref[h, :][None, :]
        g = jnp.where(g > 0, g, jnp.exp(jnp.minimum(g, 0.0)) - 1.0)
        acc = acc + jnp.dot(g, w2_ref[h * HID:(h + 1) * HID, :],
                            preferred_element_type=jnp.float32)
    h2_ref[...] = acc
    a2s_ref[...] = jnp.dot(acc, as2_ref[...], preferred_element_type=jnp.float32)
    a2d_ref[...] = jnp.dot(acc, ad2_ref[...], preferred_element_type=jnp.float32)


def _k6_body(a_ref, b2_ref, o_ref):
    num = a_ref[0, :, 1:4] + a_ref[1, :, 1:4]
    den = a_ref[0, :, 0:1] + a_ref[1, :, 0:1]
    o_ref[...] = num / (den + 1e-16) + b2_ref[...]


def kernel(x, edge_index, W1, att_src1, att_dst1, b1, W2, att_src2, att_dst2, b2):
    f32 = jnp.float32
    loop = jnp.arange(N, dtype=jnp.int32)
    src = jnp.concatenate([edge_index[0].astype(jnp.int32), loop])
    dst = jnp.concatenate([edge_index[1].astype(jnp.int32), loop])
    npad = E_PAD - E_TOT
    srcp = jnp.concatenate([src, jnp.zeros((npad,), jnp.int32)])
    dstp = jnp.concatenate([dst, jnp.full((npad,), N, jnp.int32)])
    dstg = jnp.concatenate([dst, jnp.zeros((npad,), jnp.int32)])

    eye8 = jnp.repeat(jnp.eye(HEADS, dtype=f32), HID, axis=0)
    as1m = eye8 * att_src1.reshape(HEADS * HID, 1)
    ad1m = eye8 * att_dst1.reshape(HEADS * HID, 1)

    k1 = pl.pallas_call(
        _k1_body,
        grid=(GRID,),
        in_specs=[
            pl.BlockSpec((NB, x.shape[1]), lambda i: (i, 0)),
            pl.BlockSpec((x.shape[1], HEADS * HID), lambda i: (0, 0)),
            pl.BlockSpec((HEADS * HID, HEADS), lambda i: (0, 0)),
            pl.BlockSpec((HEADS * HID, HEADS), lambda i: (0, 0)),
        ],
        out_specs=[
            pl.BlockSpec((NB, HEADS * HID), lambda i: (i, 0)),
            pl.BlockSpec((NB, HEADS), lambda i: (i, 0)),
            pl.BlockSpec((NB, HEADS), lambda i: (i, 0)),
        ],
        out_shape=[
            jax.ShapeDtypeStruct((N, HEADS * HID), f32),
            jax.ShapeDtypeStruct((N, HEADS), f32),
            jax.ShapeDtypeStruct((N, HEADS), f32),
        ],
    )
    h1, a_s1, a_d1 = k1(x, W1, as1m, ad1m)

    k2 = _make_edge_kernel(HEADS, with_msg=False)
    den1 = k2(srcp, dstg, dstp, a_s1, a_d1)[:, :N, :]

    k3 = _make_edge_kernel(HID, with_msg=True)
    msums = []
    for h in range(HEADS):
        ta = jnp.broadcast_to(a_s1[:, h:h + 1], (N, HID))
        tb = jnp.broadcast_to(a_d1[:, h:h + 1], (N, HID))
        tm = h1[:, h * HID:(h + 1) * HID]
        msums.append(k3(srcp, dstg, dstp, ta, tb, tm)[:, :N, :])
    mstack = jnp.stack(msums, axis=0)

    k4 = pl.pallas_call(
        _k4_body,
        grid=(GRID,),
        in_specs=[
            pl.BlockSpec((HEADS, 2, NB, HID), lambda i: (0, 0, i, 0)),
            pl.BlockSpec((2, NB, HEADS), lambda i: (0, i, 0)),
            pl.BlockSpec((HEADS, HID), lambda i: (0, 0)),
            pl.BlockSpec((HEADS * HID, 3), lambda i: (0, 0)),
            pl.BlockSpec((3, 1), lambda i: (0, 0)),
            pl.BlockSpec((3, 1), lambda i: (0, 0)),
        ],
        out_specs=[
            pl.BlockSpec((NB, 3), lambda i: (i, 0)),
            pl.BlockSpec((NB, 1), lambda i: (i, 0)),
            pl.BlockSpec((NB, 1), lambda i: (i, 0)),
        ],
        out_shape=[
            jax.ShapeDtypeStruct((N, 3), f32),
            jax.ShapeDtypeStruct((N, 1), f32),
            jax.ShapeDtypeStruct((N, 1), f32),
        ],
    )
    h2, a2s, a2d = k4(mstack, den1, b1.reshape(HEADS, HID), W2,
                      att_src2.reshape(3, 1), att_dst2.reshape(3, 1))

    t_a = jnp.broadcast_to(a2s, (N, 8))
    t_b = jnp.broadcast_to(a2d, (N, 8))
    t_m = jnp.concatenate(
        [jnp.ones((N, 1), f32), h2, jnp.zeros((N, 4), f32)], axis=1)

    k5 = _make_edge_kernel(8, with_msg=True)
    acc2 = k5(srcp, dstg, dstp, t_a, t_b, t_m)[:, :N, :]

    k6 = pl.pallas_call(
        _k6_body,
        grid=(GRID,),
        in_specs=[
            pl.BlockSpec((2, NB, 8), lambda i: (0, i, 0)),
            pl.BlockSpec((1, 3), lambda i: (0, 0)),
        ],
        out_specs=pl.BlockSpec((NB, 3), lambda i: (i, 0)),
        out_shape=jax.ShapeDtypeStruct((N, 3), f32),
    )
    return k6(acc2, b2.reshape(1, 3))


# speedup vs baseline: 6.3881x; 1.5442x over previous
import functools

import jax
import jax.numpy as jnp
from jax import lax
from jax.experimental import pallas as pl
from jax.experimental.pallas import tpu as pltpu
from jax.experimental.pallas import tpu_sc as plsc

N = 10000
E_RAW = 160000
E_TOT = E_RAW + N
NW = 32
CH = 64
CHUNKS = -(-E_TOT // (NW * CH))
E_PAD = NW * CH * CHUNKS
ACC_ROWS = 10240
ZB = ACC_ROWS // 16
HEADS = 8
HID = 128
NB = 400
GRID = N // NB


def _mesh():
    return plsc.VectorSubcoreMesh(core_axis_name="c", subcore_axis_name="s",
                                  num_cores=2, num_subcores=16)


def _make_edge_kernel(D, with_msg):
    ch = 128 if D <= 16 else 64
    chunks = E_PAD // (NW * ch)

    def body(src_h, dstg_h, dst_h, ta_h, tb_h, tm_h, out_h,
             isrc, idg, idst, av, bv, mv, val, acc, sem):
        c = lax.axis_index("c")
        s = lax.axis_index("s")
        wid = s * 2 + c
        slab = max(1, 256 // D)

        def zfill(r, carry):
            val[pl.ds(r * slab, slab), :] = jnp.zeros((slab, D), jnp.float32)
            return carry

        lax.fori_loop(0, ch // slab, zfill, 0)

        def zero_strip(j, carry):
            pltpu.sync_copy(val, acc.at[pl.ds(s * ZB + j * ch, ch)])
            return carry

        lax.fori_loop(0, ZB // ch, zero_strip, 0)
        plsc.subcore_barrier()

        def chunk(k, carry):
            off = (wid * chunks + k) * ch
            pltpu.sync_copy(src_h.at[pl.ds(off, ch)], isrc)
            pltpu.sync_copy(dstg_h.at[pl.ds(off, ch)], idg)
            pltpu.sync_copy(dst_h.at[pl.ds(off, ch)], idst)
            ca = pltpu.async_copy(ta_h.at[isrc], av, sem)
            cb = pltpu.async_copy(tb_h.at[idg], bv, sem)
            if with_msg:
                cm = pltpu.async_copy(tm_h.at[isrc], mv, sem)
            ca.wait()
            cb.wait()
            if with_msg:
                cm.wait()

            def compute(r, carry2):
                rs = pl.ds(r * slab, slab)
                a = av[rs, :] + bv[rs, :]
                w = jnp.exp(jnp.where(a >= 0, a, 0.2 * a))
                if with_msg:
                    val[rs, :] = w * mv[rs, :]
                else:
                    val[rs, :] = w
                return carry2

            lax.fori_loop(0, ch // slab, compute, 0)
            pltpu.sync_copy(val, acc.at[idst], add=True)
            return carry

        lax.fori_loop(0, chunks, chunk, 0)
        plsc.subcore_barrier()

        @pl.when(s == 0)
        def _():
            pltpu.sync_copy(acc, out_h.at[c])

    def no_msg_body(src_h, dstg_h, dst_h, ta_h, tb_h, out_h,
                    isrc, idg, idst, av, bv, val, acc, sem):
        body(src_h, dstg_h, dst_h, ta_h, tb_h, None, out_h,
             isrc, idg, idst, av, bv, None, val, acc, sem)

    scratch = [
        pltpu.VMEM((ch,), jnp.int32),
        pltpu.VMEM((ch,), jnp.int32),
        pltpu.VMEM((ch,), jnp.int32),
        pltpu.VMEM((ch, D), jnp.float32),
        pltpu.VMEM((ch, D), jnp.float32),
    ]
    if with_msg:
        scratch.append(pltpu.VMEM((ch, D), jnp.float32))
    scratch += [
        pltpu.VMEM((ch, D), jnp.float32),
        pltpu.VMEM_SHARED((ACC_ROWS, D), jnp.float32),
        pltpu.SemaphoreType.DMA,
    ]
    return pl.kernel(
        body if with_msg else no_msg_body,
        out_type=jax.ShapeDtypeStruct((2, ACC_ROWS, D), jnp.float32),
        mesh=_mesh(),
        scratch_types=scratch,
        compiler_params=pltpu.CompilerParams(use_tc_tiling_on_sc=False),
    )


def _k1_body(x_ref, w_ref, am_ref, bm_ref, h_ref, as_ref, ad_ref):
    h = jnp.dot(x_ref[...], w_ref[...], preferred_element_type=jnp.float32)
    h_ref[...] = h
    as_ref[...] = jnp.dot(h, am_ref[...], preferred_element_type=jnp.float32)
    ad_ref[...] = jnp.dot(h, bm_ref[...], preferred_element_type=jnp.float32)


def _k4_body(m_ref, d_ref, b1_ref, w2_ref, as2_ref, ad2_ref,
             h2_ref, a2s_ref, a2d_ref):
    den = d_ref[0] + d_ref[1]
    acc = jnp.zeros((NB, 3), jnp.float32)
    for h in range(HEADS):
        msum = m_ref[h, 0] + m_ref[h, 1]
        g = msum / (den[:, h:h + 1] + 1e-16) + b1_ref[h, :][None, :]
        g = jnp.where(g > 0, g, jnp.exp(jnp.minimum(g, 0.0)) - 1.0)
        acc = acc + jnp.dot(g, w2_ref[h * HID:(h + 1) * HID, :],
                            preferred_element_type=jnp.float32)
    h2_ref[...] = acc
    a2s_ref[...] = jnp.dot(acc, as2_ref[...], preferred_element_type=jnp.float32)
    a2d_ref[...] = jnp.dot(acc, ad2_ref[...], preferred_element_type=jnp.float32)


def _k6_body(a_ref, b2_ref, o_ref):
    num = a_ref[0, :, 1:4] + a_ref[1, :, 1:4]
    den = a_ref[0, :, 0:1] + a_ref[1, :, 0:1]
    o_ref[...] = num / (den + 1e-16) + b2_ref[...]


def kernel(x, edge_index, W1, att_src1, att_dst1, b1, W2, att_src2, att_dst2, b2):
    f32 = jnp.float32
    loop = jnp.arange(N, dtype=jnp.int32)
    src = jnp.concatenate([edge_index[0].astype(jnp.int32), loop])
    dst = jnp.concatenate([edge_index[1].astype(jnp.int32), loop])
    npad = E_PAD - E_TOT
    srcp = jnp.concatenate([src, jnp.zeros((npad,), jnp.int32)])
    dstp = jnp.concatenate([dst, jnp.full((npad,), N, jnp.int32)])
    dstg = jnp.concatenate([dst, jnp.zeros((npad,), jnp.int32)])

    eye8 = jnp.repeat(jnp.eye(HEADS, dtype=f32), HID, axis=0)
    as1m = eye8 * att_src1.reshape(HEADS * HID, 1)
    ad1m = eye8 * att_dst1.reshape(HEADS * HID, 1)

    k1 = pl.pallas_call(
        _k1_body,
        grid=(GRID,),
        in_specs=[
            pl.BlockSpec((NB, x.shape[1]), lambda i: (i, 0)),
            pl.BlockSpec((x.shape[1], HEADS * HID), lambda i: (0, 0)),
            pl.BlockSpec((HEADS * HID, HEADS), lambda i: (0, 0)),
            pl.BlockSpec((HEADS * HID, HEADS), lambda i: (0, 0)),
        ],
        out_specs=[
            pl.BlockSpec((NB, HEADS * HID), lambda i: (i, 0)),
            pl.BlockSpec((NB, HEADS), lambda i: (i, 0)),
            pl.BlockSpec((NB, HEADS), lambda i: (i, 0)),
        ],
        out_shape=[
            jax.ShapeDtypeStruct((N, HEADS * HID), f32),
            jax.ShapeDtypeStruct((N, HEADS), f32),
            jax.ShapeDtypeStruct((N, HEADS), f32),
        ],
    )
    h1, a_s1, a_d1 = k1(x, W1, as1m, ad1m)

    k2 = _make_edge_kernel(HEADS, with_msg=False)
    den1 = k2(srcp, dstg, dstp, a_s1, a_d1)[:, :N, :]

    k3 = _make_edge_kernel(HID, with_msg=True)
    msums = []
    for h in range(HEADS):
        ta = jnp.broadcast_to(a_s1[:, h:h + 1], (N, HID))
        tb = jnp.broadcast_to(a_d1[:, h:h + 1], (N, HID))
        tm = h1[:, h * HID:(h + 1) * HID]
        msums.append(k3(srcp, dstg, dstp, ta, tb, tm)[:, :N, :])
    mstack = jnp.stack(msums, axis=0)

    k4 = pl.pallas_call(
        _k4_body,
        grid=(GRID,),
        in_specs=[
            pl.BlockSpec((HEADS, 2, NB, HID), lambda i: (0, 0, i, 0)),
            pl.BlockSpec((2, NB, HEADS), lambda i: (0, i, 0)),
            pl.BlockSpec((HEADS, HID), lambda i: (0, 0)),
            pl.BlockSpec((HEADS * HID, 3), lambda i: (0, 0)),
            pl.BlockSpec((3, 1), lambda i: (0, 0)),
            pl.BlockSpec((3, 1), lambda i: (0, 0)),
        ],
        out_specs=[
            pl.BlockSpec((NB, 3), lambda i: (i, 0)),
            pl.BlockSpec((NB, 1), lambda i: (i, 0)),
            pl.BlockSpec((NB, 1), lambda i: (i, 0)),
        ],
        out_shape=[
            jax.ShapeDtypeStruct((N, 3), f32),
            jax.ShapeDtypeStruct((N, 1), f32),
            jax.ShapeDtypeStruct((N, 1), f32),
        ],
    )
    h2, a2s, a2d = k4(mstack, den1, b1.reshape(HEADS, HID), W2,
                      att_src2.reshape(3, 1), att_dst2.reshape(3, 1))

    t_a = jnp.broadcast_to(a2s, (N, 8))
    t_b = jnp.broadcast_to(a2d, (N, 8))
    t_m = jnp.concatenate(
        [jnp.ones((N, 1), f32), h2, jnp.zeros((N, 4), f32)], axis=1)

    k5 = _make_edge_kernel(8, with_msg=True)
    acc2 = k5(srcp, dstg, dstp, t_a, t_b, t_m)[:, :N, :]

    k6 = pl.pallas_call(
        _k6_body,
        grid=(GRID,),
        in_specs=[
            pl.BlockSpec((2, NB, 8), lambda i: (0, i, 0)),
            pl.BlockSpec((1, 3), lambda i: (0, 0)),
        ],
        out_specs=pl.BlockSpec((NB, 3), lambda i: (i, 0)),
        out_shape=jax.ShapeDtypeStruct((N, 3), f32),
    )
    return k6(acc2, b2.reshape(1, 3))
